# Initial kernel scaffold; baseline (speedup 1.0000x reference)
import jax
import jax.numpy as jnp
from jax.experimental import pallas as pl


def kernel(x, edge_attr, pos, params, edge_index, batch):
    raise NotImplementedError("write your pallas kernel here")


# speedup vs baseline: 1.2684x; 1.2684x over previous
import functools
import numpy as np
import jax
import jax.numpy as jnp
from jax import lax
from jax.experimental import pallas as pl
from jax.experimental.pallas import tpu as pltpu

F32 = jnp.float32


def _swish(v):
    return v * jax.nn.sigmoid(v)


def _ln(v, eps=1e-5):
    mu = jnp.mean(v, axis=-1, keepdims=True)
    var = jnp.mean((v - mu) * (v - mu), axis=-1, keepdims=True)
    return (v - mu) / jnp.sqrt(var + eps)


def _dot(a, b):
    return jnp.dot(a, b, preferred_element_type=F32)


def _prep_body(x_ref, nW_ref, nb_ref, Wms_ref, Wmd_ref, h1_ref, A_ref, B_ref):
    x = x_ref[...]
    h1 = _dot(x, nW_ref[...]) + nb_ref[...]
    h1_ref[...] = h1
    A_ref[...] = _dot(h1, Wms_ref[...])
    B_ref[...] = _dot(h1, Wmd_ref[...])


def _tc_prep(x, nW, nb, Wms, Wmd, nb_rows=1000):
    n = x.shape[0]
    grid = (n // nb_rows,)
    full = lambda shp: pl.BlockSpec(shp, lambda i: (0, 0))
    blk = lambda w: pl.BlockSpec((nb_rows, w), lambda i: (i, 0))
    return pl.pallas_call(
        _prep_body,
        grid=grid,
        in_specs=[blk(128), full((128, 128)), full((1, 128)),
                  full((128, 128)), full((128, 128))],
        out_specs=[blk(128), blk(128), blk(128)],
        out_shape=[jax.ShapeDtypeStruct((n, 128), F32)] * 3,
    )(x, nW, nb, Wms, Wmd)


def _passA_body(ga_ref, gb_ref, ea_ref, W16_ref, wd_ref, cm_ref, m_ref, dist_ref):
    ga = ga_ref[...]
    gb = gb_ref[...]
    rel = gb[:, 128:144] - ga[:, 128:144]
    d2 = jnp.sum(rel * rel, axis=1, keepdims=True)
    dist = jnp.sqrt(d2 + 1e-12)
    pre = (ga[:, :128] + gb[:, :128] + _dot(ea_ref[...], W16_ref[...])
           + dist * wd_ref[...] + cm_ref[...])
    m_ref[...] = _swish(pre)
    dist_ref[...] = dist


def _tc_pass_a(ga, gb, ea, W16, wd, cm, eb=5000):
    e = ga.shape[0]
    grid = (e // eb,)
    full = lambda shp: pl.BlockSpec(shp, lambda i: (0, 0))
    blk = lambda w: pl.BlockSpec((eb, w), lambda i: (i, 0))
    return pl.pallas_call(
        _passA_body,
        grid=grid,
        in_specs=[blk(144), blk(144), blk(16), full((16, 128)),
                  full((1, 128)), full((1, 128))],
        out_specs=[blk(128), blk(1)],
        out_shape=[jax.ShapeDtypeStruct((e, 128), F32),
                   jax.ShapeDtypeStruct((e, 1), F32)],
    )(ga, gb, ea, W16, wd, cm)


def _node_body(h_ref, aa_ref, ab_ref, Wh_ref, Wa_ref, bn_ref, o_ref):
    h = h_ref[...]
    agg = aa_ref[...] + ab_ref[...]
    u = _dot(h, Wh_ref[...]) + _dot(agg, Wa_ref[...]) + bn_ref[...]
    o_ref[...] = _ln(h + _swish(u))


def _tc_node(h, aa, ab, Wh, Wa, bn, nb_rows=1000):
    n = h.shape[0]
    grid = (n // nb_rows,)
    full = lambda shp: pl.BlockSpec(shp, lambda i: (0, 0))
    blk = lambda w: pl.BlockSpec((nb_rows, w), lambda i: (i, 0))
    return pl.pallas_call(
        _node_body,
        grid=grid,
        in_specs=[blk(128), blk(128), blk(128), full((128, 128)),
                  full((128, 128)), full((1, 128))],
        out_specs=blk(128),
        out_shape=jax.ShapeDtypeStruct((n, 128), F32),
    )(h, aa, ab, Wh, Wa, bn)


def _passB_body(gs_ref, gd_ref, ea_ref, dist_ref,
                Wes_ref, Wed_ref, W16e_ref, ce_ref, P_ref, c0_ref,
                Wm2s_ref, Wm2d_ref, Wm2e_ref, wd2_ref, cm2_ref, m2_ref):
    gs = gs_ref[...]
    gd = gd_ref[...]
    ea = ea_ref[...]
    dist = dist_ref[...]
    se = _swish(_dot(gs, Wes_ref[...]) + _dot(gd, Wed_ref[...])
                + _dot(ea, W16e_ref[...]) + ce_ref[...])
    e0 = _dot(ea, P_ref[...]) + c0_ref[...]
    e1 = _ln(e0 + se)
    pre = (_dot(gs, Wm2s_ref[...]) + _dot(gd, Wm2d_ref[...])
           + _dot(e1, Wm2e_ref[...]) + dist * wd2_ref[...] + cm2_ref[...])
    m2_ref[...] = _swish(pre)


def _tc_pass_b(gs, gd, ea, dist, Wes, Wed, W16e, ce, P, c0,
               Wm2s, Wm2d, Wm2e, wd2, cm2, eb=5000):
    e = gs.shape[0]
    grid = (e // eb,)
    full = lambda shp: pl.BlockSpec(shp, lambda i: (0, 0))
    blk = lambda w: pl.BlockSpec((eb, w), lambda i: (i, 0))
    return pl.pallas_call(
        _passB_body,
        grid=grid,
        in_specs=[blk(128), blk(128), blk(16), blk(1),
                  full((128, 128)), full((128, 128)), full((16, 128)),
                  full((1, 128)), full((16, 128)), full((1, 128)),
                  full((128, 128)), full((128, 128)), full((128, 128)),
                  full((1, 128)), full((1, 128))],
        out_specs=blk(128),
        out_shape=jax.ShapeDtypeStruct((e, 128), F32),
    )(gs, gd, ea, dist, Wes, Wed, W16e, ce, P, c0, Wm2s, Wm2d, Wm2e, wd2, cm2)


def _final_body(k_arr, h_ref, aa_ref, ab_ref, Wh_ref, Wa_ref, bn_ref,
                oW_ref, ob_ref, p_ref, out_ref, g_ref):
    h = h_ref[...]
    agg = aa_ref[...] + ab_ref[...]
    u = _dot(h, Wh_ref[...]) + _dot(agg, Wa_ref[...]) + bn_ref[...]
    h3 = _ln(h + _swish(u))
    out = _dot(h3, oW_ref[...]) + ob_ref[...]
    out_ref[...] = out

    p = p_ref[...]
    pn = p / jnp.sqrt(jnp.sum(p * p) + 1e-12)
    score = jnp.sum(out * pn, axis=1, keepdims=True)
    k = k_arr

    b = lax.bitcast_convert_type(score, jnp.int32)
    keys = jnp.where(b >= 0, b ^ jnp.int32(-2147483648), ~b).astype(jnp.uint32)

    def step(i, t):
        sh = (jnp.uint32(31) - i.astype(jnp.uint32))
        cand = t | lax.shift_left(jnp.uint32(1), sh)
        c = jnp.sum((keys >= cand).astype(jnp.int32))
        return jnp.where(c >= k, cand, t)

    t = lax.fori_loop(0, 32, step, jnp.uint32(0))
    gt = keys > t
    eq = keys == t
    cgt = jnp.sum(gt.astype(jnp.int32))
    ceq = jnp.sum(eq.astype(jnp.int32))
    r = (k - cgt).astype(F32)
    w = out * jnp.tanh(score)
    s_gt = jnp.sum(jnp.where(gt, w, 0.0), axis=0, keepdims=True)
    s_eq = jnp.sum(jnp.where(eq, w, 0.0), axis=0, keepdims=True)
    wsum = s_gt + s_eq * (r / jnp.maximum(ceq, 1).astype(F32))
    g_ref[...] = wsum / jnp.float32(k)


def _tc_final(h2, aa, ab, Wh, Wa, bn, oW, ob, p, k):
    n = h2.shape[0]
    full2 = lambda a: pl.BlockSpec(a.shape, lambda: tuple(0 for _ in a.shape))
    args = (h2, aa, ab, Wh, Wa, bn, oW, ob, p)
    return pl.pallas_call(
        functools.partial(_final_body, k),
        in_specs=[full2(a) for a in args],
        out_specs=[pl.BlockSpec((n, 64), lambda: (0, 0)),
                   pl.BlockSpec((1, 64), lambda: (0, 0))],
        out_shape=[jax.ShapeDtypeStruct((n, 64), F32),
                   jax.ShapeDtypeStruct((1, 64), F32)],
    )(*args)


def kernel(x, edge_attr, pos, params, edge_index, batch):
    n = x.shape[0]
    e = edge_attr.shape[0]
    src, dst = edge_index[0], edge_index[1]

    eW, ebias = params["edge_W"], params["edge_b"]
    L0, L1 = params["layers"]
    Wm1, bm1 = L0["Wm"], L0["bm"]
    Wn1, bn1 = L0["Wn"], L0["bn"]
    We1, be1 = L0["We"], L0["be"]
    Wm2, bm2 = L1["Wm"], L1["bm"]
    Wn2, bn2 = L1["Wn"], L1["bn"]

    row = lambda v: v[None, :]

    W16m1 = jnp.concatenate([Wm1[256:259], eW @ Wm1[259:384]], axis=0)
    cm1 = row(ebias @ Wm1[259:384] + bm1)
    W16e1 = jnp.concatenate([We1[256:259], eW @ We1[259:384]], axis=0)
    ce1 = row(ebias @ We1[259:384] + be1)
    P = jnp.zeros((16, 128), F32).at[0, 0].set(1.0).at[1, 1].set(1.0) \
        .at[2, 2].set(1.0).at[3:, 3:].set(eW)
    c0 = row(jnp.concatenate([jnp.zeros((3,), F32), ebias]))
    wd1 = row(Wm1[384])
    wd2 = row(Wm2[384])

    h1, A1, B1 = _tc_prep(x, params["node_W"], row(params["node_b"]),
                          Wm1[:128], Wm1[128:256])

    pos16 = jnp.zeros((n, 16), F32).at[:, :3].set(pos)
    A144 = jnp.concatenate([A1, pos16], axis=1)
    B144 = jnp.concatenate([B1, pos16], axis=1)

    ga = jnp.take(A144, src, axis=0)
    gb = jnp.take(B144, dst, axis=0)

    m1, dist = _tc_pass_a(ga, gb, edge_attr, W16m1, wd1, cm1)

    agg1 = jax.ops.segment_sum(m1, dst, num_segments=n)
    z = jnp.zeros_like(agg1)

    h2 = _tc_node(h1, agg1, z, Wn1[:128], Wn1[128:], row(bn1))

    gs = jnp.take(h2, src, axis=0)
    gd = jnp.take(h2, dst, axis=0)

    m2 = _tc_pass_b(gs, gd, edge_attr, dist,
                    We1[:128], We1[128:256], W16e1, ce1, P, c0,
                    Wm2[:128], Wm2[128:256], Wm2[256:384], wd2, row(bm2))

    agg2 = jax.ops.segment_sum(m2, dst, num_segments=n)

    k = int(np.ceil(0.5 * n))
    out, gemb = _tc_final(h2, agg2, z, Wn2[:128], Wn2[128:], row(bn2),
                          params["out_W"], row(params["out_b"]),
                          row(params["pool_p"]), k)
    return out, gemb


# speedup vs baseline: 3.1714x; 2.5004x over previous
import functools
import numpy as np
import jax
import jax.numpy as jnp
from jax import lax
from jax.experimental import pallas as pl
from jax.experimental.pallas import tpu as pltpu
from jax.experimental.pallas import tpu_sc as plsc

F32 = jnp.float32
NWORK = 32
CHUNK = 80


def _swish(v):
    return v * jax.nn.sigmoid(v)


def _ln(v, eps=1e-5):
    mu = jnp.mean(v, axis=-1, keepdims=True)
    var = jnp.mean((v - mu) * (v - mu), axis=-1, keepdims=True)
    return (v - mu) / jnp.sqrt(var + eps)


def _dot(a, b):
    return jnp.dot(a, b, preferred_element_type=F32)


def _prep_body(x_ref, nW_ref, nb_ref, Wms_ref, Wmd_ref, h1_ref, A_ref, B_ref):
    x = x_ref[...]
    h1 = _dot(x, nW_ref[...]) + nb_ref[...]
    h1_ref[...] = h1
    A_ref[...] = _dot(h1, Wms_ref[...])
    B_ref[...] = _dot(h1, Wmd_ref[...])


def _tc_prep(x, nW, nb, Wms, Wmd, nb_rows=1000):
    n = x.shape[0]
    grid = (n // nb_rows,)
    full = lambda shp: pl.BlockSpec(shp, lambda i: (0, 0))
    blk = lambda w: pl.BlockSpec((nb_rows, w), lambda i: (i, 0))
    return pl.pallas_call(
        _prep_body,
        grid=grid,
        in_specs=[blk(128), full((128, 128)), full((1, 128)),
                  full((128, 128)), full((128, 128))],
        out_specs=[blk(128), blk(128), blk(128)],
        out_shape=[jax.ShapeDtypeStruct((n, 128), F32)] * 3,
    )(x, nW, nb, Wms, Wmd)


def _bf(v):
    return v.astype(jnp.bfloat16).astype(F32)


def _passA_body(ga_ref, gb_ref, ev_ref, es_ref, d2_ref, eW_ref, ebias_ref,
                Wme_ref, wd_ref, bm_ref, m_ref, dist_ref):
    dist = jnp.sqrt(d2_ref[...] + 1e-12)
    e0 = jnp.concatenate([ev_ref[...], _dot(es_ref[...], eW_ref[...])
                          + ebias_ref[...]], axis=-1)
    pre = (ga_ref[...] + gb_ref[...] + _dot(e0, Wme_ref[...])
           + _bf(dist) * _bf(wd_ref[...]) + bm_ref[...])
    m_ref[...] = _swish(pre)
    dist_ref[...] = dist


def _tc_pass_a(ga, gb, ev, es, d2, eW, ebias, Wme, wd, bm, eb=3200):
    e = ga.shape[0]
    grid = (e // eb,)
    full = lambda a: pl.BlockSpec(a.shape, lambda i: (0,) * a.ndim)
    blk = lambda w: pl.BlockSpec((eb, w), lambda i: (i, 0))
    return pl.pallas_call(
        _passA_body,
        grid=grid,
        in_specs=[blk(128), blk(128), blk(3), blk(13), blk(1), full(eW),
                  full(ebias), full(Wme), full(wd), full(bm)],
        out_specs=[blk(128), blk(1)],
        out_shape=[jax.ShapeDtypeStruct((e, 128), F32),
                   jax.ShapeDtypeStruct((e, 1), F32)],
    )(ga, gb, ev, es, d2, eW, ebias, Wme, wd, bm)


def _node_body(h_ref, aa_ref, ab_ref, Wh_ref, Wa_ref, bn_ref, o_ref):
    h = h_ref[...]
    agg = aa_ref[...] + ab_ref[...]
    u = _dot(h, Wh_ref[...]) + _dot(agg, Wa_ref[...]) + bn_ref[...]
    o_ref[...] = _ln(h + _swish(u))


def _tc_node(h, aa, ab, Wh, Wa, bn, nb_rows=1000):
    n = h.shape[0]
    grid = (n // nb_rows,)
    full = lambda shp: pl.BlockSpec(shp, lambda i: (0, 0))
    blk = lambda w: pl.BlockSpec((nb_rows, w), lambda i: (i, 0))
    return pl.pallas_call(
        _node_body,
        grid=grid,
        in_specs=[blk(128), blk(128), blk(128), full((128, 128)),
                  full((128, 128)), full((1, 128))],
        out_specs=blk(128),
        out_shape=jax.ShapeDtypeStruct((n, 128), F32),
    )(h, aa, ab, Wh, Wa, bn)


def _passB_body(gs_ref, gd_ref, ev_ref, es_ref, dist_ref, eW_ref, ebias_ref,
                Wes_ref, Wed_ref, Wee_ref, be_ref,
                Wm2s_ref, Wm2d_ref, Wm2e_ref, wd2_ref, bm2_ref, m2_ref):
    gs = gs_ref[...]
    gd = gd_ref[...]
    dist = dist_ref[...]
    e0 = jnp.concatenate([ev_ref[...], _dot(es_ref[...], eW_ref[...])
                          + ebias_ref[...]], axis=-1)
    se = _swish(_dot(gs, Wes_ref[...]) + _dot(gd, Wed_ref[...])
                + _dot(e0, Wee_ref[...]) + be_ref[...])
    e1 = _ln(e0 + se)
    pre = (_dot(gs, Wm2s_ref[...]) + _dot(gd, Wm2d_ref[...])
           + _dot(e1, Wm2e_ref[...]) + _bf(dist) * _bf(wd2_ref[...])
           + bm2_ref[...])
    m2_ref[...] = _swish(pre)


def _tc_pass_b(gs, gd, ev, es, dist, eW, ebias, Wes, Wed, Wee, be,
               Wm2s, Wm2d, Wm2e, wd2, bm2, eb=2000):
    e = gs.shape[0]
    grid = (e // eb,)
    full = lambda a: pl.BlockSpec(a.shape, lambda i: (0,) * a.ndim)
    blk = lambda w: pl.BlockSpec((eb, w), lambda i: (i, 0))
    return pl.pallas_call(
        _passB_body,
        grid=grid,
        in_specs=[blk(128), blk(128), blk(3), blk(13), blk(1),
                  full(eW), full(ebias),
                  full(Wes), full(Wed), full(Wee), full(be),
                  full(Wm2s), full(Wm2d), full(Wm2e), full(wd2), full(bm2)],
        out_specs=blk(128),
        out_shape=jax.ShapeDtypeStruct((e, 128), F32),
    )(gs, gd, ev, es, dist, eW, ebias, Wes, Wed, Wee, be,
      Wm2s, Wm2d, Wm2e, wd2, bm2)


def _final_body(k_arr, h_ref, aa_ref, ab_ref, Wh_ref, Wa_ref, bn_ref,
                oW_ref, ob_ref, p_ref, out_ref, g_ref):
    h = h_ref[...]
    agg = aa_ref[...] + ab_ref[...]
    u = _dot(h, Wh_ref[...]) + _dot(agg, Wa_ref[...]) + bn_ref[...]
    h3 = _ln(h + _swish(u))
    out = _dot(h3, oW_ref[...]) + ob_ref[...]
    out_ref[...] = out

    p = p_ref[...]
    score = _dot(out, p) / jnp.sqrt(jnp.sum(p * p) + 1e-12)
    k = k_arr

    b = lax.bitcast_convert_type(score, jnp.int32)
    keys = jnp.where(b >= 0, b ^ jnp.int32(-2147483648), ~b).astype(jnp.uint32)

    def step(i, t):
        sh = (jnp.uint32(31) - i.astype(jnp.uint32))
        cand = t | lax.shift_left(jnp.uint32(1), sh)
        c = jnp.sum((keys >= cand).astype(jnp.int32))
        return jnp.where(c >= k, cand, t)

    t = lax.fori_loop(0, 32, step, jnp.uint32(0))
    gt = keys > t
    eq = keys == t
    cgt = jnp.sum(gt.astype(jnp.int32))
    ceq = jnp.sum(eq.astype(jnp.int32))
    r = (k - cgt).astype(F32)
    w = out * jnp.tanh(score)
    s_gt = jnp.sum(jnp.where(gt, w, 0.0), axis=0, keepdims=True)
    s_eq = jnp.sum(jnp.where(eq, w, 0.0), axis=0, keepdims=True)
    wsum = s_gt + s_eq * (r / jnp.maximum(ceq, 1).astype(F32))
    g_ref[...] = wsum / jnp.float32(k)


def _tc_final(h2, aa, ab, Wh, Wa, bn, oW, ob, p, k):
    n = h2.shape[0]
    full2 = lambda a: pl.BlockSpec(a.shape, lambda: tuple(0 for _ in a.shape))
    args = (h2, aa, ab, Wh, Wa, bn, oW, ob, p)
    return pl.pallas_call(
        functools.partial(_final_body, k),
        in_specs=[full2(a) for a in args],
        out_specs=[pl.BlockSpec((n, 64), lambda: (0, 0)),
                   pl.BlockSpec((1, 64), lambda: (0, 0))],
        out_shape=[jax.ShapeDtypeStruct((n, 64), F32),
                   jax.ShapeDtypeStruct((1, 64), F32)],
    )(*args)


def _sc_gather2(ta, tb, src, dst, pos3=None):
    e = src.shape[0]
    d = ta.shape[1]
    n = ta.shape[0]
    epw = e // NWORK
    n_ch = epw // CHUNK
    with_d2 = pos3 is not None
    mesh = plsc.VectorSubcoreMesh(core_axis_name="c", subcore_axis_name="s")

    out_type = [jax.ShapeDtypeStruct((e, d), F32)] * 2
    scratch = [
        pltpu.VMEM((CHUNK,), jnp.int32),
        pltpu.VMEM((CHUNK,), jnp.int32),
        pltpu.VMEM((CHUNK, d), F32),
        pltpu.VMEM((CHUNK, d), F32),
        pltpu.SemaphoreType.DMA,
        pltpu.SemaphoreType.DMA,
    ]
    if with_d2:
        out_type = out_type + [jax.ShapeDtypeStruct((e,), F32)]
        scratch = scratch + [pltpu.VMEM((6, CHUNK), F32),
                             pltpu.VMEM((CHUNK,), F32),
                             pltpu.SemaphoreType.DMA]

    @functools.partial(pl.kernel, mesh=mesh, out_type=out_type,
                       scratch_types=scratch)
    def k(*refs):
        if with_d2:
            (ta_hbm, tb_hbm, src_hbm, dst_hbm, px_hbm, py_hbm, pz_hbm,
             oa_hbm, ob_hbm, d2_hbm, idxa_v, idxb_v, rowsa_v, rowsb_v,
             sema, semb, pv_v, d2_v, semp) = refs
        else:
            (ta_hbm, tb_hbm, src_hbm, dst_hbm, oa_hbm, ob_hbm,
             idxa_v, idxb_v, rowsa_v, rowsb_v, sema, semb) = refs
        wid = lax.axis_index("c") * 16 + lax.axis_index("s")
        base = wid * epw

        def body(ci, carry):
            off = base + ci * CHUNK
            pltpu.sync_copy(src_hbm.at[pl.ds(off, CHUNK)], idxa_v)
            pltpu.sync_copy(dst_hbm.at[pl.ds(off, CHUNK)], idxb_v)
            ca = pltpu.async_copy(ta_hbm.at[idxa_v], rowsa_v, sema)
            cb = pltpu.async_copy(tb_hbm.at[idxb_v], rowsb_v, semb)
            if with_d2:
                cps = []
                for a, (p_hbm, idx) in enumerate([
                        (px_hbm, idxa_v), (py_hbm, idxa_v), (pz_hbm, idxa_v),
                        (px_hbm, idxb_v), (py_hbm, idxb_v), (pz_hbm, idxb_v)]):
                    cps.append(pltpu.async_copy(p_hbm.at[idx], pv_v.at[a], semp))
                for cp in cps:
                    cp.wait()
                for j in range(CHUNK // 16):
                    sl = pl.ds(j * 16, 16)
                    acc = jnp.zeros((16,), F32)
                    for a in range(3):
                        rel = pv_v[a + 3, sl] - pv_v[a, sl]
                        acc = acc + rel * rel
                    d2_v[sl] = acc
                pltpu.sync_copy(d2_v, d2_hbm.at[pl.ds(off, CHUNK)])
            ca.wait()
            cb.wait()
            pltpu.sync_copy(rowsa_v, oa_hbm.at[pl.ds(off, CHUNK)])
            pltpu.sync_copy(rowsb_v, ob_hbm.at[pl.ds(off, CHUNK)])
            return carry

        lax.fori_loop(0, n_ch, body, 0)

    if with_d2:
        return k(ta, tb, src, dst, pos3[0], pos3[1], pos3[2])
    return k(ta, tb, src, dst)


def _sc_scatter(m, dst, zrows, n):
    e = m.shape[0]
    epw = e // NWORK
    n_ch = epw // CHUNK
    n_nc = n // CHUNK
    mesh = plsc.VectorSubcoreMesh(core_axis_name="c", subcore_axis_name="s")

    @functools.partial(
        pl.kernel, mesh=mesh,
        out_type=jax.ShapeDtypeStruct((2, n, 128), F32),
        scratch_types=[
            pltpu.VMEM((CHUNK,), jnp.int32),
            pltpu.VMEM((CHUNK, 128), F32),
            pltpu.VMEM((CHUNK, 128), F32),
            pltpu.VMEM_SHARED((n, 128), F32),
            pltpu.SemaphoreType.DMA,
        ],
    )
    def k(m_hbm, dst_hbm, z_hbm, out_hbm, idx_v, rows_v, oc_v, acc_sh, sem):
        c = lax.axis_index("c")
        s = lax.axis_index("s")
        wid = c * 16 + s
        base = wid * epw
        cnt = (n_nc - 1 - s) // 16 + 1

        pltpu.sync_copy(z_hbm, oc_v)

        def zbody(t, carry):
            pltpu.sync_copy(oc_v, acc_sh.at[pl.ds((s + t * 16) * CHUNK, CHUNK)])
            return carry

        lax.fori_loop(0, cnt, zbody, 0)
        plsc.subcore_barrier()

        def body(ci, carry):
            off = base + ci * CHUNK
            pltpu.sync_copy(dst_hbm.at[pl.ds(off, CHUNK)], idx_v)
            pltpu.sync_copy(m_hbm.at[pl.ds(off, CHUNK)], rows_v)
            pltpu.sync_copy(rows_v, acc_sh.at[idx_v], add=True)
            return carry

        lax.fori_loop(0, n_ch, body, 0)
        plsc.subcore_barrier()

        def obody(t, carry):
            r0 = (s + t * 16) * CHUNK
            pltpu.sync_copy(acc_sh.at[pl.ds(r0, CHUNK)], oc_v)
            pltpu.sync_copy(oc_v, out_hbm.at[c, pl.ds(r0, CHUNK)])
            return carry

        lax.fori_loop(0, cnt, obody, 0)

    return k(m, dst, zrows)


def kernel(x, edge_attr, pos, params, edge_index, batch):
    n = x.shape[0]
    e = edge_attr.shape[0]
    src, dst = edge_index[0], edge_index[1]

    eW, ebias = params["edge_W"], params["edge_b"]
    L0, L1 = params["layers"]
    Wm1, bm1 = L0["Wm"], L0["bm"]
    Wn1, bn1 = L0["Wn"], L0["bn"]
    We1, be1 = L0["We"], L0["be"]
    Wm2, bm2 = L1["Wm"], L1["bm"]
    Wn2, bn2 = L1["Wn"], L1["bn"]

    row = lambda v: v[None, :]
    ev = edge_attr[:, :3]
    es = edge_attr[:, 3:]
    wd1 = row(Wm1[384])
    wd2 = row(Wm2[384])

    h1, A1, B1 = _tc_prep(x, params["node_W"], row(params["node_b"]),
                          Wm1[:128], Wm1[128:256])

    pos3 = pos.T
    ga, gb, d2 = _sc_gather2(A1, B1, src, dst, pos3=pos3)

    m1, dist = _tc_pass_a(ga, gb, ev, es, d2[:, None], eW, row(ebias),
                          Wm1[256:384], wd1, row(bm1))

    zrows = jnp.zeros((CHUNK, 128), F32)
    agg1 = _sc_scatter(m1, dst, zrows, n)

    h2 = _tc_node(h1, agg1[0], agg1[1], Wn1[:128], Wn1[128:], row(bn1))

    gs, gd = _sc_gather2(h2, h2, src, dst)

    m2 = _tc_pass_b(gs, gd, ev, es, dist, eW, row(ebias),
                    We1[:128], We1[128:256], We1[256:384], row(be1),
                    Wm2[:128], Wm2[128:256], Wm2[256:384], wd2, row(bm2))

    agg2 = _sc_scatter(m2, dst, zrows, n)

    k = int(np.ceil(0.5 * n))
    out, gemb = _tc_final(h2, agg2[0], agg2[1], Wn2[:128], Wn2[128:], row(bn2),
                          params["out_W"], row(params["out_b"]),
                          params["pool_p"][:, None], k)
    return out, gemb
